# baseline (speedup 1.0000x reference)
import jax
import jax.numpy as jnp
from jax.experimental import pallas as pl
from jax.experimental.pallas import tpu as pltpu


def _copy_body(u_ref, i_ref, out_i_ref, out_u_ref):
    out_i_ref[...] = i_ref[...]
    out_u_ref[...] = u_ref[...]


def kernel(embed_user, embed_item):
    n, d = embed_item.shape
    w = 128
    rows = n * d // w
    u = embed_user.reshape(rows, w)
    it = embed_item.reshape(rows, w)
    block = rows // 10
    grid = (rows // block,)
    out_shape = (
        jax.ShapeDtypeStruct((rows, w), embed_item.dtype),
        jax.ShapeDtypeStruct((rows, w), embed_user.dtype),
    )
    spec = pl.BlockSpec((block, w), lambda i: (i, 0))
    out_i, out_u = pl.pallas_call(
        _copy_body,
        grid=grid,
        out_shape=out_shape,
        in_specs=[spec, spec],
        out_specs=(spec, spec),
        compiler_params=pltpu.CompilerParams(
            dimension_semantics=("arbitrary",),
        ),
    )(u, it)
    return out_i.reshape(n, d), out_u.reshape(n, d)


# speedup vs baseline: 1.3180x; 1.3180x over previous
import jax
import jax.numpy as jnp
from jax.experimental import pallas as pl
from jax.experimental.pallas import tpu as pltpu

_NROW = 100000
_D = 64
_CH = 2000
_NC = _NROW // _CH
_TOTAL = 2 * _NC
_NBUF = 8
_H = 4


def _copy_body(u_hbm, i_hbm, oi_hbm, ou_hbm, bufs, in_sems, out_sems):
    ins = (i_hbm, u_hbm)
    outs = (oi_hbm, ou_hbm)

    def in_copy(k):
        t, c = k % 2, k // 2
        return pltpu.make_async_copy(
            ins[t].at[pl.ds(c * _CH, _CH), :],
            bufs.at[k % _NBUF],
            in_sems.at[k % _NBUF],
        )

    def out_copy(k):
        t, c = k % 2, k // 2
        return pltpu.make_async_copy(
            bufs.at[k % _NBUF],
            outs[t].at[pl.ds(c * _CH, _CH), :],
            out_sems.at[k % _NBUF],
        )

    for k in range(_H):
        in_copy(k).start()
    outs_waited = 0
    for k in range(_TOTAL):
        in_copy(k).wait()
        out_copy(k).start()
        nk = k + _H
        if nk < _TOTAL:
            old = nk - _NBUF
            if old >= 0:
                out_copy(old).wait()
                outs_waited = old + 1
            in_copy(nk).start()
    for k in range(outs_waited, _TOTAL):
        out_copy(k).wait()


def kernel(embed_user, embed_item):
    out_shape = (
        jax.ShapeDtypeStruct(embed_item.shape, embed_item.dtype),
        jax.ShapeDtypeStruct(embed_user.shape, embed_user.dtype),
    )
    return pl.pallas_call(
        _copy_body,
        out_shape=out_shape,
        in_specs=[
            pl.BlockSpec(memory_space=pl.ANY),
            pl.BlockSpec(memory_space=pl.ANY),
        ],
        out_specs=(
            pl.BlockSpec(memory_space=pl.ANY),
            pl.BlockSpec(memory_space=pl.ANY),
        ),
        scratch_shapes=[
            pltpu.VMEM((_NBUF, _CH, _D), jnp.float32),
            pltpu.SemaphoreType.DMA((_NBUF,)),
            pltpu.SemaphoreType.DMA((_NBUF,)),
        ],
    )(embed_user, embed_item)
